# Initial kernel scaffold; baseline (speedup 1.0000x reference)
import jax
import jax.numpy as jnp
from jax.experimental import pallas as pl


def kernel(box_features, phrase_embed, target_id, W1_sim, b1_sim, W2_sim, b2_sim, W1_reg, b1_reg, W2_reg, b2_reg, W1_sim_topN, b1_sim_topN, W2_sim_topN, b2_sim_topN, W1_reg_topN, b1_reg_topN, W2_reg_topN, b2_reg_topN):
    raise NotImplementedError("write your pallas kernel here")


# speedup vs baseline: 7.2727x; 7.2727x over previous
import functools

import jax
import jax.numpy as jnp
from jax.experimental import pallas as pl
from jax.experimental.pallas import tpu as pltpu

_K = 8


def _dot(a, b):
    return jax.lax.dot_general(
        a, b, (((1,), (0,)), ((), ())),
        precision=jax.lax.Precision.HIGHEST,
        preferred_element_type=jnp.float32)


def _lanref_kernel(tgt_ref, box_ref, phr_ref,
                   w1s_ref, b1s_ref, w2s_ref, b2s_ref,
                   w1ts_ref, b1ts_ref, w2ts_ref, b2ts_ref,
                   w1tr_ref, b1tr_ref, w2tr_ref, b2tr_ref,
                   sim_out, det_out, reg_out,
                   *, B, P, N, D_REC, D_PHR):
    f32 = jnp.float32

    phr2d = phr_ref[...].reshape(B * P, D_PHR)
    rowid = [jnp.full((1, 1), tgt_ref[b] + b * P, jnp.int32) for b in range(B)]
    rowid = jnp.concatenate(rowid, axis=0)
    iota_bp = jax.lax.broadcasted_iota(jnp.int32, (B, B * P), 1)
    oh_p = (iota_bp == rowid).astype(f32)
    phr_t = _dot(oh_p, phr2d)

    w1ts_box, w1ts_phr = w1ts_ref[0:D_REC, :], w1ts_ref[D_REC:, :]
    w1tr_box, w1tr_phr = w1tr_ref[0:D_REC, :], w1tr_ref[D_REC:, :]

    hp_ts = _dot(phr_t, w1ts_phr)
    hp_tr = _dot(phr_t, w1tr_phr)

    iota_col = jax.lax.broadcasted_iota(jnp.int32, (N, 1), 0)
    iota_kn = jax.lax.broadcasted_iota(jnp.int32, (_K, N), 1)

    sim_cols, det_cols = [], []
    for b in range(B):
        pair = jnp.concatenate(
            [box_ref[b], jnp.broadcast_to(phr_t[b:b + 1, :], (N, D_PHR))],
            axis=1)
        h = jnp.dot(pair, w1s_ref[...],
                    preferred_element_type=f32) + b1s_ref[...]
        h = jnp.where(h > 0, h, 0.01 * h)
        sim_col = jnp.dot(h, w2s_ref[...],
                          preferred_element_type=f32) + b2s_ref[...]
        sim_cols.append(sim_col)

        work = sim_col
        vals, ids = [], []
        for _ in range(_K):
            m = jnp.max(work, axis=0, keepdims=True)
            idx = jnp.min(jnp.where(work == m, iota_col, N),
                          axis=0, keepdims=True)
            vals.append(m)
            ids.append(idx)
            work = jnp.where(iota_col == idx, -jnp.inf, work)
        topv = jnp.concatenate(vals, axis=0)
        topi = jnp.concatenate(ids, axis=0)

        oh = (iota_kn == topi).astype(f32)
        box_top = _dot(oh, box_ref[b])

        h2s = _dot(box_top, w1ts_box) + hp_ts[b:b + 1, :] + b1ts_ref[...]
        h2s = jnp.where(h2s > 0, h2s, 0.01 * h2s)
        sim2 = _dot(h2s, w2ts_ref[...]) + b2ts_ref[...]

        h2r = _dot(box_top, w1tr_box) + hp_tr[b:b + 1, :] + b1tr_ref[...]
        h2r = jnp.where(h2r > 0, h2r, 0.01 * h2r)
        reg2 = _dot(h2r, w2tr_ref[...]) + b2tr_ref[...]
        reg_out[b] = reg2

        fused = sim2 * topv
        det = jnp.full((N, 1), -1e9, f32)
        for k in range(_K):
            det = jnp.where(iota_col == topi[k:k + 1, :],
                            fused[k:k + 1, :], det)
        det_cols.append(det)

    sim_out[...] = jnp.concatenate(sim_cols, axis=1)
    det_out[...] = jnp.concatenate(det_cols, axis=1)


def kernel(box_features, phrase_embed, target_id,
           W1_sim, b1_sim, W2_sim, b2_sim,
           W1_reg, b1_reg, W2_reg, b2_reg,
           W1_sim_topN, b1_sim_topN, W2_sim_topN, b2_sim_topN,
           W1_reg_topN, b1_reg_topN, W2_reg_topN, b2_reg_topN):
    del W1_reg, b1_reg, W2_reg, b2_reg
    B, N, D_REC = box_features.shape
    _, P, D_PHR = phrase_embed.shape
    f32 = jnp.float32

    vm = pl.BlockSpec(memory_space=pltpu.VMEM)
    sim_t, det_t, reg = pl.pallas_call(
        functools.partial(_lanref_kernel, B=B, P=P, N=N, D_REC=D_REC,
                          D_PHR=D_PHR),
        in_specs=[pl.BlockSpec(memory_space=pltpu.SMEM)] + [vm] * 14,
        out_specs=[vm, vm, vm],
        out_shape=[
            jax.ShapeDtypeStruct((N, B), f32),
            jax.ShapeDtypeStruct((N, B), f32),
            jax.ShapeDtypeStruct((B, _K, 6), f32),
        ],
    )(target_id, box_features, phrase_embed,
      W1_sim, b1_sim.reshape(1, -1), W2_sim, b2_sim.reshape(1, -1),
      W1_sim_topN, b1_sim_topN.reshape(1, -1), W2_sim_topN,
      b2_sim_topN.reshape(1, -1),
      W1_reg_topN, b1_reg_topN.reshape(1, -1), W2_reg_topN,
      b2_reg_topN.reshape(1, -1))
    return sim_t.T, det_t.T, reg


# speedup vs baseline: 8.7406x; 1.2018x over previous
import functools

import jax
import jax.numpy as jnp
from jax.experimental import pallas as pl
from jax.experimental.pallas import tpu as pltpu

_K = 8


def _dot(a, b):
    return jax.lax.dot_general(
        a, b, (((1,), (0,)), ((), ())),
        precision=jax.lax.Precision.HIGHEST,
        preferred_element_type=jnp.float32)


def _lanref_kernel(tgt_ref, box_ref, phr_ref,
                   w1s_ref, b1s_ref, w2s_ref, b2s_ref,
                   w1ts_ref, b1ts_ref, w2ts_ref, b2ts_ref,
                   w1tr_ref, b1tr_ref, w2tr_ref, b2tr_ref,
                   sim_out, det_out, reg_out,
                   *, B, P, N, D_REC, D_PHR):
    f32 = jnp.float32

    phr2d = phr_ref[...].reshape(B * P, D_PHR)
    rowid = [jnp.full((1, 1), tgt_ref[b] + b * P, jnp.int32) for b in range(B)]
    rowid = jnp.concatenate(rowid, axis=0)
    iota_bp = jax.lax.broadcasted_iota(jnp.int32, (B, B * P), 1)
    oh_p = (iota_bp == rowid).astype(f32)
    phr_t = _dot(oh_p, phr2d)

    phr_rep = jnp.concatenate(
        [jnp.broadcast_to(phr_t[b:b + 1, :], (N, D_PHR)) for b in range(B)],
        axis=0)
    pair = jnp.concatenate(
        [box_ref[...].reshape(B * N, D_REC), phr_rep], axis=1)
    h = jnp.dot(pair, w1s_ref[...], preferred_element_type=f32) + b1s_ref[...]
    h = jnp.where(h > 0, h, 0.01 * h)
    sim_all = jnp.dot(h, w2s_ref[...],
                      preferred_element_type=f32) + b2s_ref[...]

    sim_nb = jnp.concatenate(
        [sim_all[b * N:(b + 1) * N, :] for b in range(B)], axis=1)
    sim_mat = jnp.transpose(sim_nb)
    sim_out[...] = sim_mat

    iota_n = jax.lax.broadcasted_iota(jnp.int32, (B, N), 1)
    work = sim_mat
    vals, ids = [], []
    for _ in range(_K):
        m = jnp.max(work, axis=1, keepdims=True)
        idx = jnp.min(jnp.where(work == m, iota_n, N),
                      axis=1, keepdims=True)
        vals.append(m)
        ids.append(idx)
        work = jnp.where(iota_n == idx, -jnp.inf, work)

    w1ts_box, w1ts_phr = w1ts_ref[0:D_REC, :], w1ts_ref[D_REC:, :]
    w1tr_box, w1tr_phr = w1tr_ref[0:D_REC, :], w1tr_ref[D_REC:, :]
    hp_ts = _dot(phr_t, w1ts_phr)
    hp_tr = _dot(phr_t, w1tr_phr)

    iota_kn = jax.lax.broadcasted_iota(jnp.int32, (_K, N), 1)
    iota_1n = jax.lax.broadcasted_iota(jnp.int32, (1, N), 1)
    det_rows = []
    for b in range(B):
        ids_b = jnp.concatenate([i[b:b + 1, :] for i in ids], axis=0)
        topv_b = jnp.concatenate([v[b:b + 1, :] for v in vals], axis=0)

        oh = (iota_kn == ids_b).astype(f32)
        box_top = _dot(oh, box_ref[b])

        h2s = _dot(box_top, w1ts_box) + hp_ts[b:b + 1, :] + b1ts_ref[...]
        h2s = jnp.where(h2s > 0, h2s, 0.01 * h2s)
        sim2 = _dot(h2s, w2ts_ref[...]) + b2ts_ref[...]

        h2r = _dot(box_top, w1tr_box) + hp_tr[b:b + 1, :] + b1tr_ref[...]
        h2r = jnp.where(h2r > 0, h2r, 0.01 * h2r)
        reg2 = _dot(h2r, w2tr_ref[...]) + b2tr_ref[...]
        reg_out[b] = reg2

        fused = sim2 * topv_b
        det_b = jnp.full((1, N), -1e9, f32)
        for k in range(_K):
            det_b = jnp.where(iota_1n == ids_b[k:k + 1, :],
                              fused[k:k + 1, :], det_b)
        det_rows.append(det_b)
    det_out[...] = jnp.concatenate(det_rows, axis=0)


def kernel(box_features, phrase_embed, target_id,
           W1_sim, b1_sim, W2_sim, b2_sim,
           W1_reg, b1_reg, W2_reg, b2_reg,
           W1_sim_topN, b1_sim_topN, W2_sim_topN, b2_sim_topN,
           W1_reg_topN, b1_reg_topN, W2_reg_topN, b2_reg_topN):
    del W1_reg, b1_reg, W2_reg, b2_reg
    B, N, D_REC = box_features.shape
    _, P, D_PHR = phrase_embed.shape
    f32 = jnp.float32

    vm = pl.BlockSpec(memory_space=pltpu.VMEM)
    sim_t, det, reg = pl.pallas_call(
        functools.partial(_lanref_kernel, B=B, P=P, N=N, D_REC=D_REC,
                          D_PHR=D_PHR),
        in_specs=[pl.BlockSpec(memory_space=pltpu.SMEM)] + [vm] * 14,
        out_specs=[vm, vm, vm],
        out_shape=[
            jax.ShapeDtypeStruct((B, N), f32),
            jax.ShapeDtypeStruct((B, N), f32),
            jax.ShapeDtypeStruct((B, _K, 6), f32),
        ],
    )(target_id, box_features, phrase_embed,
      W1_sim, b1_sim.reshape(1, -1), W2_sim, b2_sim.reshape(1, -1),
      W1_sim_topN, b1_sim_topN.reshape(1, -1), W2_sim_topN,
      b2_sim_topN.reshape(1, -1),
      W1_reg_topN, b1_reg_topN.reshape(1, -1), W2_reg_topN,
      b2_reg_topN.reshape(1, -1))
    return sim_t, det, reg
